# baseline (speedup 1.0000x reference)
import functools

import jax
import jax.numpy as jnp
from jax import lax
from jax.experimental import pallas as pl
from jax.experimental.pallas import tpu as pltpu
from jax.experimental.pallas import tpu_sc as plsc

N = 10000
E = 160000
NUM_ADJS = 11
NFEAT = 256
NHID = 512

NC = 2
NS = 16
L = 16
DC = 128
NCH = NHID // DC
CPC = NCH // NC
K_E = 128
EPS = E // NS
NIT = 80
EPS_P = NIT * K_E
TPR = NIT // 2
PH = NIT // 2
DRN = 624

NB = 1000


XCH = NFEAT // DC


def _mm1_body(y_ref, w_ref, o_ref):
    i = pl.program_id(2)

    @pl.when(i == 0)
    def _zero():
        o_ref[...] = jnp.zeros_like(o_ref)

    acc = o_ref[0]
    for c in range(XCH):
        acc = acc + jnp.dot(y_ref[0, c], w_ref[0, c],
                            preferred_element_type=jnp.float32)
    o_ref[0] = acc


def _mm1(y, w1r):
    return pl.pallas_call(
        _mm1_body,
        grid=(N // NB, NCH, NUM_ADJS),
        in_specs=[
            pl.BlockSpec((1, XCH, NB, DC), lambda nb, co, i: (i, 0, nb, 0)),
            pl.BlockSpec((1, XCH, DC, DC), lambda nb, co, i: (i, 0, 0, co)),
        ],
        out_specs=pl.BlockSpec((1, NB, DC), lambda nb, co, i: (co, nb, 0)),
        out_shape=jax.ShapeDtypeStruct((NCH, N, DC), jnp.float32),
    )(y, w1r)


def _mm2_body(h_ref, b_ref, w_ref, o_ref):
    acc = None
    for q in range(NCH):
        xq = jnp.maximum(h_ref[q] + b_ref[q], 0.0)
        wq = w_ref[0, pl.ds(q * DC, DC), :]
        d = jnp.dot(xq, wq, preferred_element_type=jnp.float32)
        acc = d if acc is None else acc + d
    o_ref[0, 0] = acc


def _mm2(h1, b1r, w2):
    return pl.pallas_call(
        _mm2_body,
        grid=(NUM_ADJS, N // NB, NCH),
        in_specs=[
            pl.BlockSpec((NCH, NB, DC), lambda i, nb, c: (0, nb, 0)),
            pl.BlockSpec((NCH, 1, DC), lambda i, nb, c: (0, 0, 0)),
            pl.BlockSpec((1, NHID, DC), lambda i, nb, c: (i, 0, c)),
        ],
        out_specs=pl.BlockSpec((1, 1, NB, DC), lambda i, nb, c: (c, i, nb, 0)),
        out_shape=jax.ShapeDtypeStruct((NCH, NUM_ADJS, N, DC), jnp.float32),
    )(h1, b1r, w2)


def _epi_body(h_ref, b_ref, o_ref):
    o_ref[...] = jnp.maximum(h_ref[0] + b_ref[0], 0.0)


def _epilogue(h2, b2r):
    return pl.pallas_call(
        _epi_body,
        grid=(N // NB, NCH),
        in_specs=[
            pl.BlockSpec((1, NB, DC), lambda nb, c: (c, nb, 0)),
            pl.BlockSpec((1, 1, DC), lambda nb, c: (c, 0, 0)),
        ],
        out_specs=pl.BlockSpec((NB, DC), lambda nb, c: (nb, c)),
        out_shape=jax.ShapeDtypeStruct((N, NHID), jnp.float32),
    )(h2, b2r)


_GDN = lax.GatherDimensionNumbers(
    offset_dims=(), collapsed_slice_dims=(0,), start_index_map=(0,))


def _bcast_lane(v16, lane):
    idx = jnp.full((L, 1), lane, dtype=jnp.int32)
    return lax.gather(v16, idx, _GDN, (1,),
                      mode=lax.GatherScatterMode.PROMISE_IN_BOUNDS)

def _scale_rows(rows, val_vm, j):
    def _g(g, carry):
        v16 = val_vm[j, pl.ds(g * L, L)]
        for e in range(L):
            b = _bcast_lane(v16, e)
            r = g * L + e
            for q in range(DC // L):
                sl = pl.ds(q * L, L)
                rows[r, sl] = rows[r, sl] * b
        return carry
    lax.fori_loop(0, K_E // L, _g, 0)


def _zero_acc(rows0, acc, sid, nrows=K_E):
    def _zb(r, carry):
        for q in range(DC // L):
            rows0[r, pl.ds(q * L, L)] = jnp.zeros((L,), jnp.float32)
        return carry
    lax.fori_loop(0, nrows, _zb, 0)
    full, rem = DRN // nrows, DRN % nrows
    for z in range(full):
        pltpu.sync_copy(rows0, acc.at[pl.ds(sid * DRN + z * nrows, nrows)])
    if rem:
        pltpu.sync_copy(rows0.at[pl.ds(0, rem)],
                        acc.at[pl.ds(sid * DRN + full * nrows, rem)])

    @pl.when(sid == NS - 1)
    def _zero_tail():
        pltpu.sync_copy(rows0.at[pl.ds(0, N - NS * DRN)],
                        acc.at[pl.ds(NS * DRN, N - NS * DRN)])


def _drain_acc(acc, out, ob, sid):
    pltpu.sync_copy(acc.at[pl.ds(sid * DRN, DRN)],
                    out.at[pl.ds(ob + sid * DRN, DRN)])

    @pl.when(sid == NS - 1)
    def _drain_tail():
        pltpu.sync_copy(acc.at[pl.ds(NS * DRN, N - NS * DRN)],
                        out.at[pl.ds(ob + NS * DRN, N - NS * DRN)])


def _pair_step(t, sbase, ebase, table, srcp, dstp, valp,
               src_vm, dst_vm, val_vm, rows0, rows1, acc, g0, g1, s0, s1):
    @pl.when(t == 0)
    def _load_and_prime():
        pltpu.sync_copy(srcp.at[pl.ds(sbase, PH)], src_vm)
        pltpu.sync_copy(dstp.at[pl.ds(ebase, PH)], dst_vm)
        pltpu.sync_copy(valp.at[pl.ds(ebase, PH)], val_vm)
        pltpu.async_copy(table.at[src_vm.at[0]], rows0, g0)
        pltpu.async_copy(table.at[src_vm.at[1]], rows1, g1)

    j0 = 2 * t
    half = jnp.where(t >= TPR // 2, PH, 0)
    r0 = j0 - half
    r1 = r0 + 1
    ihalf = jnp.where(t >= TPR // 2 - 1, PH, 0)
    ri0 = j0 + 2 - ihalf
    ri1 = ri0 + 1

    pltpu.make_async_copy(table.at[src_vm.at[0]], rows0, g0).wait()
    _scale_rows(rows0, val_vm, r0)
    pltpu.async_copy(rows0, acc.at[dst_vm.at[r0]], s0, add=True)
    pltpu.make_async_copy(table.at[src_vm.at[1]], rows1, g1).wait()
    pltpu.make_async_copy(rows0, acc.at[dst_vm.at[0]], s0).wait()

    @pl.when(t == TPR // 2 - 1)
    def _reload_dst():
        pltpu.sync_copy(srcp.at[pl.ds(sbase + PH, PH)], src_vm)

    @pl.when(t < TPR - 1)
    def _next0():
        pltpu.async_copy(table.at[src_vm.at[ri0]], rows0, g0)

    _scale_rows(rows1, val_vm, r1)
    pltpu.async_copy(rows1, acc.at[dst_vm.at[r1]], s1, add=True)
    pltpu.make_async_copy(rows1, acc.at[dst_vm.at[0]], s1).wait()

    @pl.when(t == TPR // 2 - 1)
    def _reload():
        pltpu.sync_copy(dstp.at[pl.ds(ebase + PH, PH)], dst_vm)
        pltpu.sync_copy(valp.at[pl.ds(ebase + PH, PH)], val_vm)

    @pl.when(t < TPR - 1)
    def _next1():
        pltpu.async_copy(table.at[src_vm.at[ri1]], rows1, g1)


def _agg_body(table, srcp, dstp, valp, out,
              src_vm, dst_vm, val_vm, rows0, rows1, acc,
              g0, g1, s0, s1):
    cid = lax.axis_index("c")
    sid = lax.axis_index("s")

    for cc in range(CPC):
        chunk = cid * CPC + cc
        _zero_acc(rows0, acc, sid)
        plsc.subcore_barrier()

        def _body(k, carry, chunk=chunk):
            i = k // TPR
            t = k - i * TPR
            sbase = ((chunk * NUM_ADJS + i) * NS + sid) * NIT
            ebase = (i * NS + sid) * NIT
            _pair_step(t, sbase, ebase, table, srcp, dstp, valp,
                       src_vm, dst_vm, val_vm, rows0, rows1, acc,
                       g0, g1, s0, s1)
            return carry
        lax.fori_loop(0, NUM_ADJS * TPR, _body, 0)

        plsc.subcore_barrier()
        _drain_acc(acc, out, chunk * N, sid)
        plsc.subcore_barrier()


def _agg1_body(table, srcp, dstp, valp, out,
               src_vm, dst_vm, val_vm, rows0, rows1, acc,
               g0, g1, s0, s1):
    cid = lax.axis_index("c")
    sid = lax.axis_index("s")

    def _body(k, carry):
        i = k // TPR
        t = k - i * TPR
        sbase = ((cid * NUM_ADJS + i) * NS + sid) * NIT
        ebase = (i * NS + sid) * NIT

        @pl.when(t == 0)
        def _round_start():
            plsc.subcore_barrier()

            @pl.when(k > 0)
            def _drain_prev():
                _drain_acc(acc, out, ((i - 1) * XCH + cid) * N, sid)
            _zero_acc(rows0, acc, sid)
            plsc.subcore_barrier()

        _pair_step(t, sbase, ebase, table, srcp, dstp, valp,
                   src_vm, dst_vm, val_vm, rows0, rows1, acc,
                   g0, g1, s0, s1)
        return carry
    lax.fori_loop(0, NUM_ADJS * TPR, _body, 0)

    plsc.subcore_barrier()
    _drain_acc(acc, out, ((NUM_ADJS - 1) * XCH + cid) * N, sid)
    plsc.subcore_barrier()


_SC_SCRATCH = [
    pltpu.VMEM((PH, K_E), jnp.int32),
    pltpu.VMEM((PH, K_E), jnp.int32),
    pltpu.VMEM((PH, K_E), jnp.float32),
    pltpu.VMEM((K_E, DC), jnp.float32),
    pltpu.VMEM((K_E, DC), jnp.float32),
    pltpu.VMEM_SHARED((N, DC), jnp.float32),
    pltpu.SemaphoreType.DMA,
    pltpu.SemaphoreType.DMA,
    pltpu.SemaphoreType.DMA,
    pltpu.SemaphoreType.DMA,
]

_agg = functools.partial(
    pl.kernel,
    mesh=plsc.VectorSubcoreMesh(core_axis_name="c", subcore_axis_name="s"),
    out_type=jax.ShapeDtypeStruct((NCH * N, DC), jnp.float32),
    scratch_types=list(_SC_SCRATCH),
)(_agg_body)

_agg1 = functools.partial(
    pl.kernel,
    mesh=plsc.VectorSubcoreMesh(core_axis_name="c", subcore_axis_name="s"),
    out_type=jax.ShapeDtypeStruct((NUM_ADJS * XCH * N, DC), jnp.float32),
    scratch_types=list(_SC_SCRATCH),
)(_agg1_body)


K2 = 96
NCK2 = 1152
EP2 = NCK2 * K2
PH2 = 16
TRI = NCK2 // 3


def _scale_rows2(rows, val_vm, m):
    def _g(g, carry):
        v16 = val_vm[m, pl.ds(g * L, L)]
        for e in range(L):
            b = _bcast_lane(v16, e)
            r = g * L + e
            for q in range(DC // L):
                sl = pl.ds(q * L, L)
                rows[r, sl] = rows[r, sl] * b
        return carry
    lax.fori_loop(0, K2 // L, _g, 0)


def _stream_slot(u, b, sbase, ebase, nck,
                 table, srcp, dstp, valp,
                 src_vm, dst_vm, val_vm, acc, rows_, g_, s_):
    j = 3 * u + b
    m = j - (j // (2 * PH2)) * (2 * PH2)
    rb, gb, sb = rows_[b], g_[b], s_[b]
    rp, gp, sp = rows_[(b + 2) % 3], g_[(b + 2) % 3], s_[(b + 2) % 3]

    pltpu.make_async_copy(table.at[src_vm.at[0]], rb, gb).wait()
    _scale_rows2(rb, val_vm, m)
    pltpu.async_copy(rb, acc.at[dst_vm.at[m]], sb, add=True)

    def _wait_prev():
        pltpu.make_async_copy(rp, acc.at[dst_vm.at[0]], sp).wait()

    def _issue_next():
        mi = j + 2 - ((j + 2) // (2 * PH2)) * (2 * PH2)
        pltpu.async_copy(table.at[src_vm.at[mi]], rp, gp)

    if b == 0:
        @pl.when(u > 0)
        def _w0():
            _wait_prev()
        _issue_next()
    else:
        _wait_prev()

        @pl.when(j < nck - 2)
        def _i1():
            _issue_next()

    ph = j - (j // PH2) * PH2
    @pl.when((ph == 8) & (j < nck - PH2))
    def _reload():
        q1 = (j // PH2) + 1
        hrow = (q1 - (q1 // 2) * 2) * PH2
        pltpu.sync_copy(srcp.at[pl.ds(sbase + q1 * PH2, PH2)],
                        src_vm.at[pl.ds(hrow, PH2)])
        pltpu.sync_copy(dstp.at[pl.ds(ebase + q1 * PH2, PH2)],
                        dst_vm.at[pl.ds(hrow, PH2)])
        pltpu.sync_copy(valp.at[pl.ds(ebase + q1 * PH2, PH2)],
                        val_vm.at[pl.ds(hrow, PH2)])


def _agg2_body(table, srcp, dstp, valp, out,
               src_vm, dst_vm, val_vm, rowsA, rowsB, rowsC, acc,
               gA, gB, gC, sA, sB, sC):
    cid = lax.axis_index("c")
    sid = lax.axis_index("s")
    rows_ = (rowsA, rowsB, rowsC)
    g_ = (gA, gB, gC)
    s_ = (sA, sB, sC)

    for cc in range(CPC):
        chunk = cid * CPC + cc
        _zero_acc(rowsA, acc, sid, K2)
        plsc.subcore_barrier()

        sbase = (chunk * NS + sid) * NCK2
        ebase = sid * NCK2
        pltpu.sync_copy(srcp.at[pl.ds(sbase, 2 * PH2)], src_vm)
        pltpu.sync_copy(dstp.at[pl.ds(ebase, 2 * PH2)], dst_vm)
        pltpu.sync_copy(valp.at[pl.ds(ebase, 2 * PH2)], val_vm)
        pltpu.async_copy(table.at[src_vm.at[0]], rowsA, gA)
        pltpu.async_copy(table.at[src_vm.at[1]], rowsB, gB)

        def _body(u, carry):
            for b in range(3):
                _stream_slot(u, b, sbase, ebase, NCK2,
                             table, srcp, dstp, valp,
                             src_vm, dst_vm, val_vm, acc, rows_, g_, s_)
            return carry
        lax.fori_loop(0, TRI, _body, 0)

        lb = (NCK2 - 1) % 3
        pltpu.make_async_copy(rows_[lb], acc.at[dst_vm.at[0]], s_[lb]).wait()

        plsc.subcore_barrier()
        _drain_acc(acc, out, chunk * N, sid)
        plsc.subcore_barrier()


NCK1 = 108
NCK1A = 120
UPR1 = NCK1 // 3


def _agg1s_body(table, srcp, dstp, valp, out,
                src_vm, dst_vm, val_vm, rowsA, rowsB, rowsC, acc,
                gA, gB, gC, sA, sB, sC):
    cid = lax.axis_index("c")
    sid = lax.axis_index("s")
    rows_ = (rowsA, rowsB, rowsC)
    g_ = (gA, gB, gC)
    s_ = (sA, sB, sC)
    lb = (NCK1 - 1) % 3

    def _body(k, carry):
        i = k // UPR1
        u = k - i * UPR1
        sbase = ((cid * NUM_ADJS + i) * NS + sid) * NCK1A
        ebase = (i * NS + sid) * NCK1A

        @pl.when(u == 0)
        def _round_start():
            @pl.when(k > 0)
            def _fin_prev():
                pltpu.make_async_copy(rows_[lb], acc.at[dst_vm.at[0]],
                                      s_[lb]).wait()
            plsc.subcore_barrier()

            @pl.when(k > 0)
            def _drain_prev():
                _drain_acc(acc, out, ((i - 1) * XCH + cid) * N, sid)
            _zero_acc(rowsA, acc, sid, K2)
            plsc.subcore_barrier()
            pltpu.sync_copy(srcp.at[pl.ds(sbase, 2 * PH2)], src_vm)
            pltpu.sync_copy(dstp.at[pl.ds(ebase, 2 * PH2)], dst_vm)
            pltpu.sync_copy(valp.at[pl.ds(ebase, 2 * PH2)], val_vm)
            pltpu.async_copy(table.at[src_vm.at[0]], rowsA, gA)
            pltpu.async_copy(table.at[src_vm.at[1]], rowsB, gB)

        for b in range(3):
            _stream_slot(u, b, sbase, ebase, NCK1,
                         table, srcp, dstp, valp,
                         src_vm, dst_vm, val_vm, acc, rows_, g_, s_)
        return carry
    lax.fori_loop(0, NUM_ADJS * UPR1, _body, 0)

    pltpu.make_async_copy(rows_[lb], acc.at[dst_vm.at[0]], s_[lb]).wait()
    plsc.subcore_barrier()
    _drain_acc(acc, out, ((NUM_ADJS - 1) * XCH + cid) * N, sid)
    plsc.subcore_barrier()


_SC2_SCRATCH = [
    pltpu.VMEM((2 * PH2, K2), jnp.int32),
    pltpu.VMEM((2 * PH2, K2), jnp.int32),
    pltpu.VMEM((2 * PH2, K2), jnp.float32),
    pltpu.VMEM((K2, DC), jnp.float32),
    pltpu.VMEM((K2, DC), jnp.float32),
    pltpu.VMEM((K2, DC), jnp.float32),
    pltpu.VMEM_SHARED((N, DC), jnp.float32),
    pltpu.SemaphoreType.DMA,
    pltpu.SemaphoreType.DMA,
    pltpu.SemaphoreType.DMA,
    pltpu.SemaphoreType.DMA,
    pltpu.SemaphoreType.DMA,
    pltpu.SemaphoreType.DMA,
]

_agg1s = functools.partial(
    pl.kernel,
    mesh=plsc.VectorSubcoreMesh(core_axis_name="c", subcore_axis_name="s"),
    out_type=jax.ShapeDtypeStruct((NUM_ADJS * XCH * N, DC), jnp.float32),
    scratch_types=list(_SC2_SCRATCH),
)(_agg1s_body)


_agg2 = functools.partial(
    pl.kernel,
    mesh=plsc.VectorSubcoreMesh(core_axis_name="c", subcore_axis_name="s"),
    out_type=jax.ShapeDtypeStruct((NCH * N, DC), jnp.float32),
    scratch_types=[
        pltpu.VMEM((2 * PH2, K2), jnp.int32),
        pltpu.VMEM((2 * PH2, K2), jnp.int32),
        pltpu.VMEM((2 * PH2, K2), jnp.float32),
        pltpu.VMEM((K2, DC), jnp.float32),
        pltpu.VMEM((K2, DC), jnp.float32),
        pltpu.VMEM((K2, DC), jnp.float32),
        pltpu.VMEM_SHARED((N, DC), jnp.float32),
        pltpu.SemaphoreType.DMA,
        pltpu.SemaphoreType.DMA,
        pltpu.SemaphoreType.DMA,
        pltpu.SemaphoreType.DMA,
        pltpu.SemaphoreType.DMA,
        pltpu.SemaphoreType.DMA,
    ],
)(_agg2_body)


def kernel(x, adj_indices, adj_values, w1, b1, w2, b2):
    pad1 = ((0, 0), (0, 0), (0, NCK1A * K2 - EPS))
    src1 = jnp.pad(adj_indices[:, 1, :].reshape(NUM_ADJS, NS, EPS), pad1)
    off1 = jnp.arange(XCH, dtype=jnp.int32)[:, None, None, None] * N
    srcs1 = (src1[None] + off1).reshape(-1, K2)
    dsts1 = jnp.pad(adj_indices[:, 0, :].reshape(NUM_ADJS, NS, EPS),
                    pad1).reshape(-1, K2)
    vals1 = jnp.pad(adj_values.reshape(NUM_ADJS, NS, EPS),
                    pad1).reshape(-1, K2)

    pad2 = ((0, 0), (0, 0), (0, EP2 - NUM_ADJS * EPS))
    off2 = (jnp.arange(NCH, dtype=jnp.int32)[:, None] * NUM_ADJS
            + jnp.arange(NUM_ADJS, dtype=jnp.int32)[None, :]) * N
    src2 = adj_indices[:, 1, :].reshape(NUM_ADJS, NS, EPS)
    srcs = (src2[None] + off2[:, :, None, None]).transpose(0, 2, 1, 3)
    srcs = jnp.pad(srcs.reshape(NCH, NS, NUM_ADJS * EPS), pad2)
    srcs = srcs.reshape(-1, K2)
    dst2 = adj_indices[:, 0, :].reshape(NUM_ADJS, NS, EPS).transpose(1, 0, 2)
    dsts = jnp.pad(dst2.reshape(1, NS, NUM_ADJS * EPS), pad2).reshape(-1, K2)
    val2 = adj_values.reshape(NUM_ADJS, NS, EPS).transpose(1, 0, 2)
    vals = jnp.pad(val2.reshape(1, NS, NUM_ADJS * EPS), pad2).reshape(-1, K2)
    b1r = b1.reshape(NCH, 1, DC)
    b2r = b2.reshape(NCH, 1, DC)

    xt = x.reshape(N, XCH, DC).transpose(1, 0, 2).reshape(XCH * N, DC)
    y = _agg1s(xt, srcs1, dsts1, vals1).reshape(NUM_ADJS, XCH, N, DC)
    w1r = w1.reshape(NUM_ADJS, XCH, DC, NHID)
    h1 = _mm1(y, w1r)
    t2 = _mm2(h1, b1r, w2).reshape(NCH * NUM_ADJS * N, DC)
    h2 = _agg2(t2, srcs, dsts, vals).reshape(NCH, N, DC)
    return _epilogue(h2, b2r)


# speedup vs baseline: 1.1001x; 1.1001x over previous
import functools

import jax
import jax.numpy as jnp
from jax import lax
from jax.experimental import pallas as pl
from jax.experimental.pallas import tpu as pltpu
from jax.experimental.pallas import tpu_sc as plsc

N = 10000
E = 160000
NUM_ADJS = 11
NFEAT = 256
NHID = 512

NC = 2
NS = 16
L = 16
DC = 128
NCH = NHID // DC
CPC = NCH // NC
K_E = 128
EPS = E // NS
NIT = 80
EPS_P = NIT * K_E
TPR = NIT // 2
PH = NIT // 2
DRN = 624

NB = 1000


XCH = NFEAT // DC


def _mm1_body(y_ref, w_ref, o_ref):
    i = pl.program_id(2)

    @pl.when(i == 0)
    def _zero():
        o_ref[...] = jnp.zeros_like(o_ref)

    acc = o_ref[0]
    for c in range(XCH):
        acc = acc + jnp.dot(y_ref[0, c], w_ref[0, c],
                            preferred_element_type=jnp.float32)
    o_ref[0] = acc


def _mm1(y, w1r):
    return pl.pallas_call(
        _mm1_body,
        grid=(N // NB, NCH, NUM_ADJS),
        in_specs=[
            pl.BlockSpec((1, XCH, NB, DC), lambda nb, co, i: (i, 0, nb, 0)),
            pl.BlockSpec((1, XCH, DC, DC), lambda nb, co, i: (i, 0, 0, co)),
        ],
        out_specs=pl.BlockSpec((1, NB, DC), lambda nb, co, i: (co, nb, 0)),
        out_shape=jax.ShapeDtypeStruct((NCH, N, DC), jnp.float32),
    )(y, w1r)


def _mm2_body(h_ref, b_ref, w_ref, o_ref):
    acc = None
    for q in range(NCH):
        xq = jnp.maximum(h_ref[q] + b_ref[q], 0.0)
        wq = w_ref[0, pl.ds(q * DC, DC), :]
        d = jnp.dot(xq, wq, preferred_element_type=jnp.float32)
        acc = d if acc is None else acc + d
    o_ref[0, 0] = acc


def _mm2(h1, b1r, w2):
    return pl.pallas_call(
        _mm2_body,
        grid=(NUM_ADJS, N // NB, NCH),
        in_specs=[
            pl.BlockSpec((NCH, NB, DC), lambda i, nb, c: (0, nb, 0)),
            pl.BlockSpec((NCH, 1, DC), lambda i, nb, c: (0, 0, 0)),
            pl.BlockSpec((1, NHID, DC), lambda i, nb, c: (i, 0, c)),
        ],
        out_specs=pl.BlockSpec((1, 1, NB, DC), lambda i, nb, c: (c, i, nb, 0)),
        out_shape=jax.ShapeDtypeStruct((NCH, NUM_ADJS, N, DC), jnp.float32),
    )(h1, b1r, w2)


def _epi_body(h_ref, b_ref, o_ref):
    o_ref[...] = jnp.maximum(h_ref[0] + b_ref[0], 0.0)


def _epilogue(h2, b2r):
    return pl.pallas_call(
        _epi_body,
        grid=(N // NB, NCH),
        in_specs=[
            pl.BlockSpec((1, NB, DC), lambda nb, c: (c, nb, 0)),
            pl.BlockSpec((1, 1, DC), lambda nb, c: (c, 0, 0)),
        ],
        out_specs=pl.BlockSpec((NB, DC), lambda nb, c: (nb, c)),
        out_shape=jax.ShapeDtypeStruct((N, NHID), jnp.float32),
    )(h2, b2r)


_GDN = lax.GatherDimensionNumbers(
    offset_dims=(), collapsed_slice_dims=(0,), start_index_map=(0,))


def _bcast_lane(v16, lane):
    idx = jnp.full((L, 1), lane, dtype=jnp.int32)
    return lax.gather(v16, idx, _GDN, (1,),
                      mode=lax.GatherScatterMode.PROMISE_IN_BOUNDS)

def _scale_rows(rows, val_vm, j):
    def _g(g, carry):
        v16 = val_vm[j, pl.ds(g * L, L)]
        for e in range(L):
            b = _bcast_lane(v16, e)
            r = g * L + e
            for q in range(DC // L):
                sl = pl.ds(q * L, L)
                rows[r, sl] = rows[r, sl] * b
        return carry
    lax.fori_loop(0, K_E // L, _g, 0)


def _zero_acc(rows0, acc, sid, nrows=K_E):
    def _zb(r, carry):
        for q in range(DC // L):
            rows0[r, pl.ds(q * L, L)] = jnp.zeros((L,), jnp.float32)
        return carry
    lax.fori_loop(0, nrows, _zb, 0)
    full, rem = DRN // nrows, DRN % nrows
    for z in range(full):
        pltpu.sync_copy(rows0, acc.at[pl.ds(sid * DRN + z * nrows, nrows)])
    if rem:
        pltpu.sync_copy(rows0.at[pl.ds(0, rem)],
                        acc.at[pl.ds(sid * DRN + full * nrows, rem)])

    @pl.when(sid == NS - 1)
    def _zero_tail():
        pltpu.sync_copy(rows0.at[pl.ds(0, N - NS * DRN)],
                        acc.at[pl.ds(NS * DRN, N - NS * DRN)])


def _drain_acc(acc, out, ob, sid):
    pltpu.sync_copy(acc.at[pl.ds(sid * DRN, DRN)],
                    out.at[pl.ds(ob + sid * DRN, DRN)])

    @pl.when(sid == NS - 1)
    def _drain_tail():
        pltpu.sync_copy(acc.at[pl.ds(NS * DRN, N - NS * DRN)],
                        out.at[pl.ds(ob + NS * DRN, N - NS * DRN)])


def _pair_step(t, sbase, ebase, table, srcp, dstp, valp,
               src_vm, dst_vm, val_vm, rows0, rows1, acc, g0, g1, s0, s1):
    @pl.when(t == 0)
    def _load_and_prime():
        pltpu.sync_copy(srcp.at[pl.ds(sbase, PH)], src_vm)
        pltpu.sync_copy(dstp.at[pl.ds(ebase, PH)], dst_vm)
        pltpu.sync_copy(valp.at[pl.ds(ebase, PH)], val_vm)
        pltpu.async_copy(table.at[src_vm.at[0]], rows0, g0)
        pltpu.async_copy(table.at[src_vm.at[1]], rows1, g1)

    j0 = 2 * t
    half = jnp.where(t >= TPR // 2, PH, 0)
    r0 = j0 - half
    r1 = r0 + 1
    ihalf = jnp.where(t >= TPR // 2 - 1, PH, 0)
    ri0 = j0 + 2 - ihalf
    ri1 = ri0 + 1

    pltpu.make_async_copy(table.at[src_vm.at[0]], rows0, g0).wait()
    _scale_rows(rows0, val_vm, r0)
    pltpu.async_copy(rows0, acc.at[dst_vm.at[r0]], s0, add=True)
    pltpu.make_async_copy(table.at[src_vm.at[1]], rows1, g1).wait()
    pltpu.make_async_copy(rows0, acc.at[dst_vm.at[0]], s0).wait()

    @pl.when(t == TPR // 2 - 1)
    def _reload_dst():
        pltpu.sync_copy(srcp.at[pl.ds(sbase + PH, PH)], src_vm)

    @pl.when(t < TPR - 1)
    def _next0():
        pltpu.async_copy(table.at[src_vm.at[ri0]], rows0, g0)

    _scale_rows(rows1, val_vm, r1)
    pltpu.async_copy(rows1, acc.at[dst_vm.at[r1]], s1, add=True)
    pltpu.make_async_copy(rows1, acc.at[dst_vm.at[0]], s1).wait()

    @pl.when(t == TPR // 2 - 1)
    def _reload():
        pltpu.sync_copy(dstp.at[pl.ds(ebase + PH, PH)], dst_vm)
        pltpu.sync_copy(valp.at[pl.ds(ebase + PH, PH)], val_vm)

    @pl.when(t < TPR - 1)
    def _next1():
        pltpu.async_copy(table.at[src_vm.at[ri1]], rows1, g1)


def _agg_body(table, srcp, dstp, valp, out,
              src_vm, dst_vm, val_vm, rows0, rows1, acc,
              g0, g1, s0, s1):
    cid = lax.axis_index("c")
    sid = lax.axis_index("s")

    for cc in range(CPC):
        chunk = cid * CPC + cc
        _zero_acc(rows0, acc, sid)
        plsc.subcore_barrier()

        def _body(k, carry, chunk=chunk):
            i = k // TPR
            t = k - i * TPR
            sbase = ((chunk * NUM_ADJS + i) * NS + sid) * NIT
            ebase = (i * NS + sid) * NIT
            _pair_step(t, sbase, ebase, table, srcp, dstp, valp,
                       src_vm, dst_vm, val_vm, rows0, rows1, acc,
                       g0, g1, s0, s1)
            return carry
        lax.fori_loop(0, NUM_ADJS * TPR, _body, 0)

        plsc.subcore_barrier()
        _drain_acc(acc, out, chunk * N, sid)
        plsc.subcore_barrier()


def _agg1_body(table, srcp, dstp, valp, out,
               src_vm, dst_vm, val_vm, rows0, rows1, acc,
               g0, g1, s0, s1):
    cid = lax.axis_index("c")
    sid = lax.axis_index("s")

    def _body(k, carry):
        i = k // TPR
        t = k - i * TPR
        sbase = ((cid * NUM_ADJS + i) * NS + sid) * NIT
        ebase = (i * NS + sid) * NIT

        @pl.when(t == 0)
        def _round_start():
            plsc.subcore_barrier()

            @pl.when(k > 0)
            def _drain_prev():
                _drain_acc(acc, out, ((i - 1) * XCH + cid) * N, sid)
            _zero_acc(rows0, acc, sid)
            plsc.subcore_barrier()

        _pair_step(t, sbase, ebase, table, srcp, dstp, valp,
                   src_vm, dst_vm, val_vm, rows0, rows1, acc,
                   g0, g1, s0, s1)
        return carry
    lax.fori_loop(0, NUM_ADJS * TPR, _body, 0)

    plsc.subcore_barrier()
    _drain_acc(acc, out, ((NUM_ADJS - 1) * XCH + cid) * N, sid)
    plsc.subcore_barrier()


_SC_SCRATCH = [
    pltpu.VMEM((PH, K_E), jnp.int32),
    pltpu.VMEM((PH, K_E), jnp.int32),
    pltpu.VMEM((PH, K_E), jnp.float32),
    pltpu.VMEM((K_E, DC), jnp.float32),
    pltpu.VMEM((K_E, DC), jnp.float32),
    pltpu.VMEM_SHARED((N, DC), jnp.float32),
    pltpu.SemaphoreType.DMA,
    pltpu.SemaphoreType.DMA,
    pltpu.SemaphoreType.DMA,
    pltpu.SemaphoreType.DMA,
]

_agg = functools.partial(
    pl.kernel,
    mesh=plsc.VectorSubcoreMesh(core_axis_name="c", subcore_axis_name="s"),
    out_type=jax.ShapeDtypeStruct((NCH * N, DC), jnp.float32),
    scratch_types=list(_SC_SCRATCH),
)(_agg_body)

_agg1 = functools.partial(
    pl.kernel,
    mesh=plsc.VectorSubcoreMesh(core_axis_name="c", subcore_axis_name="s"),
    out_type=jax.ShapeDtypeStruct((NUM_ADJS * XCH * N, DC), jnp.float32),
    scratch_types=list(_SC_SCRATCH),
)(_agg1_body)


K2 = 96
NCK2 = 1152
EP2 = NCK2 * K2
PH2 = 16
TRI = NCK2 // 3


def _scale_rows2(rows, val_vm, m):
    def _g(g, carry):
        v16 = val_vm[m, pl.ds(g * L, L)]
        for e in range(L):
            b = _bcast_lane(v16, e)
            r = g * L + e
            for q in range(DC // L):
                sl = pl.ds(q * L, L)
                rows[r, sl] = rows[r, sl] * b
        return carry
    lax.fori_loop(0, K2 // L, _g, 0)


def _stream_slot(u, b, sbase, ebase, nck,
                 table, srcp, dstp, valp,
                 src_vm, dst_vm, val_vm, acc, rows_, g_, s_):
    j = 3 * u + b
    m = j - (j // (2 * PH2)) * (2 * PH2)
    rb, gb, sb = rows_[b], g_[b], s_[b]
    rp, gp, sp = rows_[(b + 2) % 3], g_[(b + 2) % 3], s_[(b + 2) % 3]

    pltpu.make_async_copy(table.at[src_vm.at[0]], rb, gb).wait()
    _scale_rows2(rb, val_vm, m)
    pltpu.async_copy(rb, acc.at[dst_vm.at[m]], sb, add=True)

    def _wait_prev():
        pltpu.make_async_copy(rp, acc.at[dst_vm.at[0]], sp).wait()

    def _issue_next():
        mi = j + 2 - ((j + 2) // (2 * PH2)) * (2 * PH2)
        pltpu.async_copy(table.at[src_vm.at[mi]], rp, gp)

    if b == 0:
        @pl.when(u > 0)
        def _w0():
            _wait_prev()
        _issue_next()
    else:
        _wait_prev()

        @pl.when(j < nck - 2)
        def _i1():
            _issue_next()

    ph = j - (j // PH2) * PH2
    @pl.when((ph == 8) & (j < nck - PH2))
    def _reload():
        q1 = (j // PH2) + 1
        hrow = (q1 - (q1 // 2) * 2) * PH2
        pltpu.sync_copy(srcp.at[pl.ds(sbase + q1 * PH2, PH2)],
                        src_vm.at[pl.ds(hrow, PH2)])
        pltpu.sync_copy(dstp.at[pl.ds(ebase + q1 * PH2, PH2)],
                        dst_vm.at[pl.ds(hrow, PH2)])
        pltpu.sync_copy(valp.at[pl.ds(ebase + q1 * PH2, PH2)],
                        val_vm.at[pl.ds(hrow, PH2)])


def _agg2_body(table, srcp, dstp, valp, out,
               src_vm, dst_vm, val_vm, rowsA, rowsB, rowsC, acc,
               gA, gB, gC, sA, sB, sC):
    cid = lax.axis_index("c")
    sid = lax.axis_index("s")
    rows_ = (rowsA, rowsB, rowsC)
    g_ = (gA, gB, gC)
    s_ = (sA, sB, sC)

    for cc in range(CPC):
        chunk = cid * CPC + cc
        _zero_acc(rowsA, acc, sid, K2)
        plsc.subcore_barrier()

        sbase = (chunk * NS + sid) * NCK2
        ebase = sid * NCK2
        pltpu.sync_copy(srcp.at[pl.ds(sbase, 2 * PH2)], src_vm)
        pltpu.sync_copy(dstp.at[pl.ds(ebase, 2 * PH2)], dst_vm)
        pltpu.sync_copy(valp.at[pl.ds(ebase, 2 * PH2)], val_vm)
        pltpu.async_copy(table.at[src_vm.at[0]], rowsA, gA)
        pltpu.async_copy(table.at[src_vm.at[1]], rowsB, gB)

        def _body(u, carry):
            for b in range(3):
                _stream_slot(u, b, sbase, ebase, NCK2,
                             table, srcp, dstp, valp,
                             src_vm, dst_vm, val_vm, acc, rows_, g_, s_)
            return carry
        lax.fori_loop(0, TRI, _body, 0)

        lb = (NCK2 - 1) % 3
        pltpu.make_async_copy(rows_[lb], acc.at[dst_vm.at[0]], s_[lb]).wait()

        plsc.subcore_barrier()
        _drain_acc(acc, out, chunk * N, sid)
        plsc.subcore_barrier()


NCK1 = 108
NCK1A = 120
UPR1 = NCK1 // 3


def _agg1s_body(table, srcp, dstp, valp, out,
                src_vm, dst_vm, val_vm, rowsA, rowsB, rowsC, acc,
                gA, gB, gC, sA, sB, sC):
    cid = lax.axis_index("c")
    sid = lax.axis_index("s")
    rows_ = (rowsA, rowsB, rowsC)
    g_ = (gA, gB, gC)
    s_ = (sA, sB, sC)
    lb = (NCK1 - 1) % 3

    def _body(k, carry):
        i = k // UPR1
        u = k - i * UPR1
        sbase = ((cid * NUM_ADJS + i) * NS + sid) * NCK1A
        ebase = (i * NS + sid) * NCK1A

        @pl.when(u == 0)
        def _round_start():
            @pl.when(k > 0)
            def _fin_prev():
                pltpu.make_async_copy(rows_[lb], acc.at[dst_vm.at[0]],
                                      s_[lb]).wait()
            plsc.subcore_barrier()

            @pl.when(k > 0)
            def _drain_prev():
                _drain_acc(acc, out, ((i - 1) * XCH + cid) * N, sid)
            _zero_acc(rowsA, acc, sid, K2)
            plsc.subcore_barrier()
            pltpu.sync_copy(srcp.at[pl.ds(sbase, 2 * PH2)], src_vm)
            pltpu.sync_copy(dstp.at[pl.ds(ebase, 2 * PH2)], dst_vm)
            pltpu.sync_copy(valp.at[pl.ds(ebase, 2 * PH2)], val_vm)
            pltpu.async_copy(table.at[src_vm.at[0]], rowsA, gA)
            pltpu.async_copy(table.at[src_vm.at[1]], rowsB, gB)

        for b in range(3):
            _stream_slot(u, b, sbase, ebase, NCK1,
                         table, srcp, dstp, valp,
                         src_vm, dst_vm, val_vm, acc, rows_, g_, s_)
        return carry
    lax.fori_loop(0, NUM_ADJS * UPR1, _body, 0)

    pltpu.make_async_copy(rows_[lb], acc.at[dst_vm.at[0]], s_[lb]).wait()
    plsc.subcore_barrier()
    _drain_acc(acc, out, ((NUM_ADJS - 1) * XCH + cid) * N, sid)
    plsc.subcore_barrier()


_SC2_SCRATCH = [
    pltpu.VMEM((2 * PH2, K2), jnp.int32),
    pltpu.VMEM((2 * PH2, K2), jnp.int32),
    pltpu.VMEM((2 * PH2, K2), jnp.float32),
    pltpu.VMEM((K2, DC), jnp.float32),
    pltpu.VMEM((K2, DC), jnp.float32),
    pltpu.VMEM((K2, DC), jnp.float32),
    pltpu.VMEM_SHARED((N, DC), jnp.float32),
    pltpu.SemaphoreType.DMA,
    pltpu.SemaphoreType.DMA,
    pltpu.SemaphoreType.DMA,
    pltpu.SemaphoreType.DMA,
    pltpu.SemaphoreType.DMA,
    pltpu.SemaphoreType.DMA,
]

_agg1s = functools.partial(
    pl.kernel,
    mesh=plsc.VectorSubcoreMesh(core_axis_name="c", subcore_axis_name="s"),
    out_type=jax.ShapeDtypeStruct((NUM_ADJS * XCH * N, DC), jnp.float32),
    scratch_types=list(_SC2_SCRATCH),
)(_agg1s_body)


_agg2 = functools.partial(
    pl.kernel,
    mesh=plsc.VectorSubcoreMesh(core_axis_name="c", subcore_axis_name="s"),
    out_type=jax.ShapeDtypeStruct((NCH * N, DC), jnp.float32),
    scratch_types=[
        pltpu.VMEM((2 * PH2, K2), jnp.int32),
        pltpu.VMEM((2 * PH2, K2), jnp.int32),
        pltpu.VMEM((2 * PH2, K2), jnp.float32),
        pltpu.VMEM((K2, DC), jnp.float32),
        pltpu.VMEM((K2, DC), jnp.float32),
        pltpu.VMEM((K2, DC), jnp.float32),
        pltpu.VMEM_SHARED((N, DC), jnp.float32),
        pltpu.SemaphoreType.DMA,
        pltpu.SemaphoreType.DMA,
        pltpu.SemaphoreType.DMA,
        pltpu.SemaphoreType.DMA,
        pltpu.SemaphoreType.DMA,
        pltpu.SemaphoreType.DMA,
    ],
)(_agg2_body)


def kernel(x, adj_indices, adj_values, w1, b1, w2, b2):
    pad1 = ((0, 0), (0, 0), (0, EPS_P - EPS))
    src1 = jnp.pad(adj_indices[:, 1, :].reshape(NUM_ADJS, NS, EPS), pad1)
    off1 = jnp.arange(XCH, dtype=jnp.int32)[:, None, None, None] * N
    srcs1 = (src1[None] + off1).reshape(-1, K_E)
    dsts1 = jnp.pad(adj_indices[:, 0, :].reshape(NUM_ADJS, NS, EPS),
                    pad1).reshape(-1, K_E)
    vals1 = jnp.pad(adj_values.reshape(NUM_ADJS, NS, EPS),
                    pad1).reshape(-1, K_E)

    pad2 = ((0, 0), (0, 0), (0, EP2 - NUM_ADJS * EPS))
    off2 = (jnp.arange(NCH, dtype=jnp.int32)[:, None] * NUM_ADJS
            + jnp.arange(NUM_ADJS, dtype=jnp.int32)[None, :]) * N
    src2 = adj_indices[:, 1, :].reshape(NUM_ADJS, NS, EPS)
    srcs = (src2[None] + off2[:, :, None, None]).transpose(0, 2, 1, 3)
    srcs = jnp.pad(srcs.reshape(NCH, NS, NUM_ADJS * EPS), pad2)
    srcs = srcs.reshape(-1, K2)
    dst2 = adj_indices[:, 0, :].reshape(NUM_ADJS, NS, EPS).transpose(1, 0, 2)
    dsts = jnp.pad(dst2.reshape(1, NS, NUM_ADJS * EPS), pad2).reshape(-1, K2)
    val2 = adj_values.reshape(NUM_ADJS, NS, EPS).transpose(1, 0, 2)
    vals = jnp.pad(val2.reshape(1, NS, NUM_ADJS * EPS), pad2).reshape(-1, K2)
    b1r = b1.reshape(NCH, 1, DC)
    b2r = b2.reshape(NCH, 1, DC)

    xt = x.reshape(N, XCH, DC).transpose(1, 0, 2).reshape(XCH * N, DC)
    y = _agg1(xt, srcs1, dsts1, vals1).reshape(NUM_ADJS, XCH, N, DC)
    w1r = w1.reshape(NUM_ADJS, XCH, DC, NHID)
    h1 = _mm1(y, w1r)
    t2 = _mm2(h1, b1r, w2).reshape(NCH * NUM_ADJS * N, DC)
    h2 = _agg2(t2, srcs, dsts, vals).reshape(NCH, N, DC)
    return _epilogue(h2, b2r)
